# baseline (speedup 1.0000x reference)
import functools

import jax
import jax.numpy as jnp
from jax import lax
from jax.experimental import pallas as pl
from jax.experimental.pallas import tpu as pltpu
from jax.experimental.pallas import tpu_sc as plsc

_L = 16


def _iota():
    return lax.iota(jnp.int32, _L)


@functools.lru_cache(maxsize=None)
def _make_phase_a(d, v):
    info = plsc.get_sparse_core_info()
    nc = info.num_cores
    nw = nc * info.num_subcores
    n_full = v // 128
    rem = v - n_full * 128
    base_cnt, extra = divmod(n_full, nw)
    max_cnt = base_cnt + (1 if extra else 0)
    n_outer = (max_cnt + 1) // 2

    mesh = plsc.VectorSubcoreMesh(core_axis_name="c", subcore_axis_name="s")

    @functools.partial(
        pl.kernel,
        mesh=mesh,
        out_type=jax.ShapeDtypeStruct((v // 2, 2 * d), jnp.float32),
        scratch_types=[
            pltpu.VMEM((2, d, 128), jnp.float32),
            pltpu.VMEM((2, d, 128), jnp.float32),
        ]
        + [pltpu.SemaphoreType.DMA] * 4,
        compiler_params=pltpu.CompilerParams(needs_layout_passes=False),
    )
    def phase_a(table_t, tail2, scratch, vbuf, tbuf,
                isem0, isem1, osem0, osem1):
        isem = (isem0, isem1)
        osem = (osem0, osem1)
        w = lax.axis_index("s") * nc + lax.axis_index("c")
        cnt = jnp.where(w < extra, base_cnt + 1, base_cnt)
        start = base_cnt * w + jnp.minimum(w, extra)
        dvec = [(_iota() + 16 * j) for j in range(4)]

        def fire_in(i, b):
            pltpu.async_copy(
                table_t.at[:, pl.ds((start + i) * 128, 128)],
                vbuf.at[b], isem[b],
            )

        def drain(sem, buf):
            pltpu.make_async_copy(
                table_t.at[:, pl.ds(0, 128)], buf, sem
            ).wait()

        def transpose_block(b, n_pairs):
            def body(u, carry):
                for gj in range(8):
                    col = jnp.full((_L,), 2 * u + (1 if gj >= 4 else 0),
                                   jnp.int32)
                    vals = plsc.load_gather(
                        vbuf.at[b], [dvec[gj % 4], col])
                    tbuf[b, u, pl.ds(16 * gj, 16)] = vals
                return carry
            lax.fori_loop(0, n_pairs, body, 0)

        def fire_out(i, b):
            pltpu.async_copy(
                tbuf.at[b],
                scratch.at[pl.ds((start + i) * (128 // 2), 128 // 2), :],
                osem[b],
            )

        fire_in(0, 0)

        def outer(k, carry):
            for b in range(2):
                i = 2 * k + b

                @pl.when(i < cnt)
                def _():
                    @pl.when(i + 1 < cnt)
                    def _():
                        fire_in(i + 1, 1 - b)

                    drain(isem[b], vbuf.at[b])

                    @pl.when(i >= 2)
                    def _():
                        drain(osem[b], tbuf.at[b])

                    transpose_block(b, 64)
                    fire_out(i, b)
            return carry

        lax.fori_loop(0, n_outer, outer, 0)
        for b in range(2):
            drain(osem[b], tbuf.at[b])

        if rem:
            @pl.when(w == nw - 1)
            def _():
                pltpu.sync_copy(tail2, vbuf.at[0, pl.ds(0, rem // 2)])
                pltpu.sync_copy(
                    vbuf.at[0, pl.ds(0, rem // 2)],
                    scratch.at[pl.ds(n_full * 64, rem // 2), :],
                )

    return phase_a


@functools.lru_cache(maxsize=None)
def _make_phase_b(d, v, hist, batch):
    info = plsc.get_sparse_core_info()
    nc = info.num_cores
    nw = nc * info.num_subcores
    n_btile = batch // 128
    cols_per_w = n_btile // nw
    units = hist * cols_per_w
    lanes_per_w = cols_per_w * 128

    mesh = plsc.VectorSubcoreMesh(core_axis_name="c", subcore_axis_name="s")

    @functools.partial(
        pl.kernel,
        mesh=mesh,
        out_type=jax.ShapeDtypeStruct((hist, d, batch), jnp.float32),
        scratch_types=[
            pltpu.VMEM((hist, lanes_per_w), jnp.int32),
            pltpu.VMEM((2, 128), jnp.int32),
            pltpu.VMEM((2, 128, 2 * d), jnp.float32),
            pltpu.VMEM((2, d, 128), jnp.float32),
        ]
        + [pltpu.SemaphoreType.DMA] * 4,
        compiler_params=pltpu.CompilerParams(needs_layout_passes=False),
    )
    def phase_b(scratch, idx_t, out, ibuf, glist, gbuf, tbuf,
                gsem0, gsem1, osem0, osem1):
        gsem = (gsem0, gsem1)
        osem = (osem0, osem1)
        w = lax.axis_index("s") * nc + lax.axis_index("c")
        base_lane = w * lanes_per_w
        pltpu.sync_copy(idx_t.at[:, pl.ds(base_lane, lanes_per_w)], ibuf)
        rows16 = [(_iota() + 16 * g) for g in range(8)]

        def unit_hj(i):
            return i // cols_per_w, lax.rem(i, cols_per_w)

        def fire_gather(i, b):
            h, jc = unit_hj(i)
            for g in range(8):
                v16 = ibuf[h, pl.ds(jc * 128 + 16 * g, 16)]
                glist[b, pl.ds(16 * g, 16)] = v16 >> 1
            pltpu.async_copy(scratch.at[glist.at[b]], gbuf.at[b], gsem[b])

        def drain(sem, buf):
            pltpu.make_async_copy(scratch.at[pl.ds(0, 128)], buf, sem).wait()

        def transpose_out(j, nb):
            h, jc = unit_hj(j)

            @pl.when(j >= 2)
            def _():
                drain(osem[nb], tbuf.at[nb])

            for g in range(8):
                par = (ibuf[h, pl.ds(jc * 128 + 16 * g, 16)] & 1) << 6

                def dbody(d8, carry):
                    for dd in range(8):
                        di = d8 * 8 + dd
                        vals = plsc.load_gather(
                            gbuf.at[nb], [rows16[g], par + di])
                        tbuf[nb, di, pl.ds(16 * g, 16)] = vals
                    return carry
                lax.fori_loop(0, 8, dbody, 0)

            pltpu.async_copy(
                tbuf.at[nb],
                out.at[h, :, pl.ds(base_lane + jc * 128, 128)],
                osem[nb],
            )

        def outer(k, carry):
            for b in range(2):
                i = 2 * k + b
                fire_gather(i, b)

                @pl.when(i >= 1)
                def _():
                    drain(gsem[1 - b], gbuf.at[1 - b])
                    transpose_out(i - 1, 1 - b)
            return carry

        lax.fori_loop(0, units // 2, outer, 0)
        last = units - 1
        nb = last % 2
        drain(gsem[nb], gbuf.at[nb])
        transpose_out(last, nb)
        for b in range(2):
            drain(osem[b], tbuf.at[b])

    return phase_b


def kernel(batch_data, item_embeddings):
    batch, hist = batch_data.shape
    v, d = item_embeddings.shape
    table_t = item_embeddings.T
    idx_t = batch_data.T.astype(jnp.int32)
    n_full = v // 128
    tail2 = item_embeddings[n_full * 128:].reshape(-1, 2 * d)
    scratch = _make_phase_a(d, v)(table_t, tail2)
    out3 = _make_phase_b(d, v, hist, batch)(scratch, idx_t)
    return out3.transpose(2, 0, 1)


# speedup vs baseline: 1.7456x; 1.7456x over previous
import functools

import jax
import jax.numpy as jnp
from jax import lax
from jax.experimental import pallas as pl
from jax.experimental.pallas import tpu as pltpu
from jax.experimental.pallas import tpu_sc as plsc

_L = 16
_PW = 136


def _iota():
    return lax.iota(jnp.int32, _L)


@functools.lru_cache(maxsize=None)
def _make_phase_a(d, v):
    info = plsc.get_sparse_core_info()
    nc = info.num_cores
    nw = nc * info.num_subcores
    n_full = v // 128
    rem = v - n_full * 128
    base_cnt, extra = divmod(n_full, nw)
    max_cnt = base_cnt + (1 if extra else 0)
    n_outer = (max_cnt + 1) // 2

    mesh = plsc.VectorSubcoreMesh(core_axis_name="c", subcore_axis_name="s")

    @functools.partial(
        pl.kernel,
        mesh=mesh,
        out_type=jax.ShapeDtypeStruct((v, 128), jnp.float32),
        scratch_types=[
            pltpu.VMEM((2, d, 128), jnp.float32),
            pltpu.VMEM((2, 128, _PW), jnp.float32),
        ]
        + [pltpu.SemaphoreType.DMA] * 4,
        compiler_params=pltpu.CompilerParams(needs_layout_passes=False),
    )
    def phase_a(table_t, tail_pad, scratch, vbuf, tbuf,
                isem0, isem1, osem0, osem1):
        isem = (isem0, isem1)
        osem = (osem0, osem1)
        w = lax.axis_index("s") * nc + lax.axis_index("c")
        cnt = jnp.where(w < extra, base_cnt + 1, base_cnt)
        start = base_cnt * w + jnp.minimum(w, extra)
        l16 = [(_iota() + 16 * l0) for l0 in range(8)]

        def fire_in(i, b):
            pltpu.async_copy(
                table_t.at[:, pl.ds((start + i) * 128, 128)],
                vbuf.at[b], isem[b],
            )

        def drain(sem, buf):
            pltpu.make_async_copy(
                table_t.at[:, pl.ds(0, 128)], buf, sem
            ).wait()

        def transpose_block(b):
            @plsc.parallel_loop(0, d, unroll=8)
            def _(dd):
                for l0 in range(8):
                    vals = vbuf[b, dd, pl.ds(16 * l0, 16)]
                    plsc.store_scatter(
                        tbuf.at[b], [l16[l0], jnp.full((_L,), 0, jnp.int32)
                                     + dd], vals)

        def fire_out(i, b):
            pltpu.async_copy(
                tbuf.at[b, :, pl.ds(0, 128)],
                scratch.at[pl.ds((start + i) * 128, 128), :],
                osem[b],
            )

        fire_in(0, 0)

        def outer(k, carry):
            for b in range(2):
                i = 2 * k + b

                @pl.when(i < cnt)
                def _():
                    @pl.when(i + 1 < cnt)
                    def _():
                        fire_in(i + 1, 1 - b)

                    drain(isem[b], vbuf.at[b])

                    @pl.when(i >= 2)
                    def _():
                        drain(osem[b], tbuf.at[b, :, pl.ds(0, 128)])

                    transpose_block(b)
                    fire_out(i, b)
            return carry

        lax.fori_loop(0, n_outer, outer, 0)
        for b in range(2):
            drain(osem[b], tbuf.at[b, :, pl.ds(0, 128)])

        if rem:
            @pl.when(w == nw - 1)
            def _():
                pltpu.sync_copy(tail_pad, vbuf.at[0, pl.ds(0, rem)])
                pltpu.sync_copy(
                    vbuf.at[0, pl.ds(0, rem)],
                    scratch.at[pl.ds(n_full * 128, rem), :],
                )

    return phase_a


@functools.lru_cache(maxsize=None)
def _make_phase_b(d, v, hist, batch):
    info = plsc.get_sparse_core_info()
    nc = info.num_cores
    nw = nc * info.num_subcores
    n_btile = batch // 128
    cols_per_w = n_btile // nw
    units = hist * cols_per_w
    lanes_per_w = cols_per_w * 128

    mesh = plsc.VectorSubcoreMesh(core_axis_name="c", subcore_axis_name="s")

    @functools.partial(
        pl.kernel,
        mesh=mesh,
        out_type=jax.ShapeDtypeStruct((hist, d, batch), jnp.float32),
        scratch_types=[
            pltpu.VMEM((hist, lanes_per_w), jnp.int32),
            pltpu.VMEM((2, 128, 128), jnp.float32),
            pltpu.VMEM((2, d, _PW), jnp.float32),
        ]
        + [pltpu.SemaphoreType.DMA] * 4,
        compiler_params=pltpu.CompilerParams(needs_layout_passes=False),
    )
    def phase_b(scratch, idx_t, out, ibuf, gbuf, tbuf,
                gsem0, gsem1, osem0, osem1):
        gsem = (gsem0, gsem1)
        osem = (osem0, osem1)
        w = lax.axis_index("s") * nc + lax.axis_index("c")
        base_lane = w * lanes_per_w
        pltpu.sync_copy(idx_t.at[:, pl.ds(base_lane, lanes_per_w)], ibuf)
        c16 = [(_iota() + 16 * cc) for cc in range(d // 16)]

        def unit_hj(i):
            return i // cols_per_w, lax.rem(i, cols_per_w)

        def fire_gather(i, b):
            h, jc = unit_hj(i)
            pltpu.async_copy(
                scratch.at[ibuf.at[h, pl.ds(jc * 128, 128)]],
                gbuf.at[b], gsem[b],
            )

        def drain(sem, buf):
            pltpu.make_async_copy(scratch.at[pl.ds(0, 128)], buf, sem).wait()

        def transpose_out(j, nb):
            h, jc = unit_hj(j)

            @pl.when(j >= 2)
            def _():
                drain(osem[nb], tbuf.at[nb, :, pl.ds(0, 128)])

            @plsc.parallel_loop(0, 128, unroll=8)
            def _(p):
                for cc in range(d // 16):
                    vals = gbuf[nb, p, pl.ds(16 * cc, 16)]
                    plsc.store_scatter(
                        tbuf.at[nb],
                        [c16[cc], jnp.full((_L,), 0, jnp.int32) + p], vals)

            pltpu.async_copy(
                tbuf.at[nb, :, pl.ds(0, 128)],
                out.at[h, :, pl.ds(base_lane + jc * 128, 128)],
                osem[nb],
            )

        def outer(k, carry):
            for b in range(2):
                i = 2 * k + b
                fire_gather(i, b)

                @pl.when(i >= 1)
                def _():
                    drain(gsem[1 - b], gbuf.at[1 - b])
                    transpose_out(i - 1, 1 - b)
            return carry

        lax.fori_loop(0, units // 2, outer, 0)
        last = units - 1
        nb = last % 2
        drain(gsem[nb], gbuf.at[nb])
        transpose_out(last, nb)
        for b in range(2):
            drain(osem[b], tbuf.at[b, :, pl.ds(0, 128)])

    return phase_b


def kernel(batch_data, item_embeddings):
    batch, hist = batch_data.shape
    v, d = item_embeddings.shape
    table_t = item_embeddings.T
    idx_t = batch_data.T.astype(jnp.int32)
    n_full = v // 128
    tail_pad = jnp.pad(item_embeddings[n_full * 128:], ((0, 0), (0, 128 - d)))
    scratch = _make_phase_a(d, v)(table_t, tail_pad)
    out3 = _make_phase_b(d, v, hist, batch)(scratch, idx_t)
    return out3.transpose(2, 0, 1)
